# baseline (speedup 1.0000x reference)
import functools

import jax
import jax.numpy as jnp
from jax import lax
from jax.experimental import pallas as pl
from jax.experimental.pallas import tpu as pltpu

N = 10000
E = 320000
D = 128
DE = 16
H = 8
C = 128
HC = H * C

ROWB = 400


def _proj_body(x_ref, wq_ref, bq_ref, wk_ref, bk_ref, wv_ref, bv_ref,
               wskip_ref, bskip_ref, bd_ref,
               q_ref, k_ref, v_ref, qe_ref, skip_ref):
    xb = x_ref[...]
    inv_sqrt_c = 1.0 / (C ** 0.5)
    q = (jnp.dot(xb, wq_ref[...], preferred_element_type=jnp.float32)
         + bq_ref[...]) * inv_sqrt_c
    q_ref[...] = q
    k_ref[...] = jnp.dot(xb, wk_ref[...], preferred_element_type=jnp.float32) + bk_ref[...]
    v_ref[...] = jnp.dot(xb, wv_ref[...], preferred_element_type=jnp.float32) + bv_ref[...]
    skip_ref[...] = jnp.dot(xb, wskip_ref[...], preferred_element_type=jnp.float32) + bskip_ref[...]
    qe_ref[...] = jnp.dot(q, bd_ref[...], preferred_element_type=jnp.float32)


def _projections(x, Wq, bq, Wk, bk, Wv, bv, Wskip, bskip, BD):
    nblk = N // ROWB
    full = lambda shape: pl.BlockSpec(shape, lambda i: (0,) * len(shape))
    row = lambda w: pl.BlockSpec((ROWB, w), lambda i: (i, 0))
    return pl.pallas_call(
        _proj_body,
        grid=(nblk,),
        in_specs=[row(D), full((D, HC)), full((1, HC)), full((D, HC)),
                  full((1, HC)), full((D, HC)), full((1, HC)),
                  full((D, C)), full((1, C)), full((HC, H * DE))],
        out_specs=[row(HC), row(HC), row(HC), row(H * DE), row(C)],
        out_shape=[
            jax.ShapeDtypeStruct((N, HC), jnp.float32),
            jax.ShapeDtypeStruct((N, HC), jnp.float32),
            jax.ShapeDtypeStruct((N, HC), jnp.float32),
            jax.ShapeDtypeStruct((N, H * DE), jnp.float32),
            jax.ShapeDtypeStruct((N, C), jnp.float32),
        ],
    )(x, Wq, bq.reshape(1, HC), Wk, bk.reshape(1, HC), Wv, bv.reshape(1, HC),
      Wskip, bskip.reshape(1, C), BD)


def _epi_body(numv_ref, aux_ref, bd2_ref, skip_ref, wd_ref, bd_ref, out_ref):
    i = pl.program_id(0)
    aux = aux_ref[...].sum(axis=0)
    acc = jnp.zeros((ROWB, C), jnp.float32)
    for h in range(H):
        wea_h = aux[:, h * 20:h * 20 + DE]
        den_h = aux[:, h * 20 + DE:h * 20 + DE + 1]
        emsg = jnp.dot(wea_h, bd2_ref[h * DE:(h + 1) * DE, h * C:(h + 1) * C],
                       preferred_element_type=jnp.float32)
        tot = numv_ref[:, h * C:(h + 1) * C] + emsg
        acc = acc + tot / (den_h + 1e-16)
    node = jnp.maximum(acc * (1.0 / H) + skip_ref[...], 0.0)
    part = jnp.dot(node, wd_ref[...], preferred_element_type=jnp.float32)
    psum = jnp.sum(part).reshape(1, 1)

    @pl.when(i == 0)
    def _():
        out_ref[...] = bd_ref[...]
    out_ref[...] += psum


def _epilogue(numv, aux, BD2, skip, Wdense, bdense):
    nblk = N // ROWB
    P = aux.shape[0]
    full = lambda shape: pl.BlockSpec(shape, lambda i: (0,) * len(shape))
    return pl.pallas_call(
        _epi_body,
        grid=(nblk,),
        in_specs=[pl.BlockSpec((ROWB, HC), lambda i: (i, 0)),
                  pl.BlockSpec((P, ROWB, H * 20), lambda i: (0, i, 0)),
                  full((H * DE, HC)),
                  pl.BlockSpec((ROWB, C), lambda i: (i, 0)),
                  full((C, 1)), full((1, 1))],
        out_specs=pl.BlockSpec((1, 1), lambda i: (0, 0)),
        out_shape=jax.ShapeDtypeStruct((1, 1), jnp.float32),
    )(numv, aux, BD2, skip, Wdense, bdense.reshape(1, 1))


def _edge_middle(src, dst, edge_attr, q, k, v, qe):
    qd = q[dst].reshape(E, H, C)
    ks = k[src].reshape(E, H, C)
    alpha = (qd * ks).sum(axis=-1)
    alpha = alpha + (qe[dst].reshape(E, H, DE) * edge_attr[:, None, :]).sum(axis=-1)
    ex = jnp.exp(alpha)
    numv = jax.ops.segment_sum(
        (ex[:, :, None] * v[src].reshape(E, H, C)).reshape(E, HC), dst,
        num_segments=N)
    wea = jax.ops.segment_sum(ex[:, :, None] * edge_attr[:, None, :], dst,
                              num_segments=N)
    denom = jax.ops.segment_sum(ex, dst, num_segments=N)
    aux = jnp.concatenate(
        [wea, denom[:, :, None],
         jnp.zeros((N, H, 3), jnp.float32)], axis=-1).reshape(1, N, H * 20)
    return numv, aux


def kernel(x, edge_index, edge_attr, Wq, bq, Wk, bk, Wv, bv, We,
           Wskip, bskip, Wdense, bdense):
    WeT = We.reshape(DE, H, C).transpose(1, 2, 0)
    eye = jnp.eye(H, dtype=jnp.float32)
    BD = (WeT[:, :, None, :] * eye[:, None, :, None]).reshape(HC, H * DE)
    BD2 = BD.T

    q, k, v, qe, skip = _projections(x, Wq, bq, Wk, bk, Wv, bv, Wskip, bskip, BD)
    src = edge_index[0]
    dst = edge_index[1]
    numv, aux = _edge_middle(src, dst, edge_attr, q, k, v, qe)
    out = _epilogue(numv, aux, BD2, skip, Wdense, bdense)
    return out.reshape(1)


# speedup vs baseline: 4.8916x; 4.8916x over previous
import functools

import jax
import jax.numpy as jnp
from jax import lax
from jax.experimental import pallas as pl
from jax.experimental.pallas import tpu as pltpu
from jax.experimental.pallas import tpu_sc as plsc

N = 10000
E = 320000
D = 128
DE = 16
H = 8
C = 128
HC = H * C

ROWB = 400

NW = 32
EPW = E // NW
NCHUNK = 16
CH = 640
RECW = 128
NVW = HC + 256
NB_ROWS = E + 16 * NCHUNK + 16
_SC_MESH = lambda: plsc.VectorSubcoreMesh(core_axis_name="c", subcore_axis_name="s")


def _lanes():
    return lax.iota(jnp.int32, 16)


def _gath(vec, idx):
    return lax.gather(
        vec, idx[:, None],
        lax.GatherDimensionNumbers(offset_dims=(), collapsed_slice_dims=(0,),
                                   start_index_map=(0,)),
        (1,), mode=lax.GatherScatterMode.PROMISE_IN_BOUNDS)


def _splat(vec, i):
    return _gath(vec, jnp.full((16,), i, jnp.int32))


def _fold_sum(v):
    lanes = _lanes()
    for kk in (1, 2, 4, 8):
        v = v + _gath(v, jnp.bitwise_xor(lanes, jnp.full((16,), kk, jnp.int32)))
    return v


def _scan_incl(v):
    lanes = _lanes()
    zero = jnp.zeros((16,), v.dtype)
    for kk in (1, 2, 4, 8):
        kv = jnp.full((16,), kk, jnp.int32)
        shifted = _gath(v, jnp.maximum(lanes - kv, jnp.zeros((16,), jnp.int32)))
        v = v + jnp.where(lanes >= kv, shifted, zero)
    return v


def _sc_count(dst):
    @functools.partial(
        pl.kernel,
        out_type=jax.ShapeDtypeStruct((NW, 16), jnp.int32),
        mesh=_SC_MESH(),
        scratch_types=[pltpu.VMEM((EPW,), jnp.int32),
                       pltpu.VMEM((16,), jnp.int32)],
    )
    def k(dst_hbm, cnt_hbm, dbuf, cntv):
        wid = lax.axis_index("s") * 2 + lax.axis_index("c")
        pltpu.sync_copy(dst_hbm.at[pl.ds(wid * EPW, EPW)], dbuf)
        lanes = _lanes()
        chc = jnp.full((16,), CH, jnp.int32)
        onev = jnp.full((16,), 1, jnp.int32)
        zerov = jnp.zeros((16,), jnp.int32)

        def body(g, accs):
            chv = lax.div(dbuf[pl.ds(g * 16, 16)], chc)
            return tuple(
                accs[ch] + jnp.where(chv == jnp.full((16,), ch, jnp.int32),
                                     onev, zerov)
                for ch in range(NCHUNK))

        accs = lax.fori_loop(0, EPW // 16, body,
                             tuple(jnp.zeros((16,), jnp.int32)
                                   for _ in range(NCHUNK)))
        out = jnp.zeros((16,), jnp.int32)
        for ch in range(NCHUNK):
            tot = _fold_sum(accs[ch])
            out = jnp.where(lanes == jnp.full((16,), ch, jnp.int32), tot, out)
        cntv[...] = out
        pltpu.sync_copy(cntv, cnt_hbm.at[wid])

    return k(dst)


def _proj_body(x_ref, wq_ref, bq_ref, wk_ref, bk_ref, wv_ref, bv_ref,
               wskip_ref, bskip_ref, bd_ref,
               q_ref, k_ref, v_ref, qe_ref, skip_ref):
    xb = x_ref[...]
    inv_sqrt_c = 1.0 / (C ** 0.5)
    q = (jnp.dot(xb, wq_ref[...], preferred_element_type=jnp.float32)
         + bq_ref[...]) * inv_sqrt_c
    q_ref[...] = q
    k_ref[...] = jnp.dot(xb, wk_ref[...], preferred_element_type=jnp.float32) + bk_ref[...]
    v_ref[...] = jnp.dot(xb, wv_ref[...], preferred_element_type=jnp.float32) + bv_ref[...]
    skip_ref[...] = jnp.dot(xb, wskip_ref[...], preferred_element_type=jnp.float32) + bskip_ref[...]
    qe_ref[...] = jnp.dot(q, bd_ref[...], preferred_element_type=jnp.float32)


def _projections(x, Wq, bq, Wk, bk, Wv, bv, Wskip, bskip, BD):
    nblk = N // ROWB
    full = lambda shape: pl.BlockSpec(shape, lambda i: (0,) * len(shape))
    row = lambda w: pl.BlockSpec((ROWB, w), lambda i: (i, 0))
    return pl.pallas_call(
        _proj_body,
        grid=(nblk,),
        in_specs=[row(D), full((D, HC)), full((1, HC)), full((D, HC)),
                  full((1, HC)), full((D, HC)), full((1, HC)),
                  full((D, C)), full((1, C)), full((HC, H * DE))],
        out_specs=[row(HC), row(HC), row(HC), row(H * DE), row(C)],
        out_shape=[
            jax.ShapeDtypeStruct((N, HC), jnp.float32),
            jax.ShapeDtypeStruct((N, HC), jnp.float32),
            jax.ShapeDtypeStruct((N, HC), jnp.float32),
            jax.ShapeDtypeStruct((N, H * DE), jnp.float32),
            jax.ShapeDtypeStruct((N, C), jnp.float32),
        ],
    )(x, Wq, bq.reshape(1, HC), Wk, bk.reshape(1, HC), Wv, bv.reshape(1, HC),
      Wskip, bskip.reshape(1, C), BD)


def _sc_edge_pass(src, dst, edge_attr, q, k, qe, offs, cse, cpe):
    @functools.partial(
        pl.kernel,
        out_type=jax.ShapeDtypeStruct((NB_ROWS, RECW), jnp.float32),
        mesh=_SC_MESH(),
        scratch_types=[
            pltpu.VMEM((EPW,), jnp.int32),
            pltpu.VMEM((EPW,), jnp.int32),
            pltpu.VMEM((16,), jnp.int32),
            pltpu.VMEM((16,), jnp.int32),
            pltpu.VMEM((16, DE), jnp.float32),
            pltpu.VMEM((16, HC), jnp.float32),
            pltpu.VMEM((16, HC), jnp.float32),
            pltpu.VMEM((16, H * DE), jnp.float32),
            pltpu.VMEM((16, RECW), jnp.float32),
            pltpu.VMEM((16,), jnp.int32),
            pltpu.VMEM((16,), jnp.int32),
        ],
    )
    def kfn(src_hbm, dst_hbm, attr_hbm, q_hbm, k_hbm, qe_hbm, offs_hbm,
            cse_hbm, cpe_hbm, buck_hbm, srcs, dsts, srcv, dstv, attrb,
            qrows, krows, qerows, recblk, posr, basesr):
        core = lax.axis_index("c")
        sub = lax.axis_index("s")
        wid = sub * 2 + core
        lanes = _lanes()
        zero16 = jnp.zeros((16,), jnp.float32)

        def zr(r, _):
            for cc in range(2, RECW // 16):
                recblk[r, pl.ds(cc * 16, 16)] = zero16
            return 0
        lax.fori_loop(0, 16, zr, 0)

        pltpu.sync_copy(src_hbm.at[pl.ds(wid * EPW, EPW)], srcs)
        pltpu.sync_copy(dst_hbm.at[pl.ds(wid * EPW, EPW)], dsts)
        pltpu.sync_copy(offs_hbm.at[wid], basesr)
        lane8 = lanes < jnp.full((16,), 8, jnp.int32)

        def group(g, _):
            base = wid * EPW + g * 16
            sv = srcs[pl.ds(g * 16, 16)]
            dv = dsts[pl.ds(g * 16, 16)]
            srcv[...] = sv
            dstv[...] = dv
            pltpu.sync_copy(attr_hbm.at[pl.ds(base, 16)], attrb)
            pltpu.sync_copy(q_hbm.at[dstv], qrows)
            pltpu.sync_copy(k_hbm.at[srcv], krows)
            pltpu.sync_copy(qe_hbm.at[dstv], qerows)

            def edge(j, _):
                attr_j = attrb[j, :]
                alpha = zero16
                for h in range(H):
                    acc = qerows[j, pl.ds(h * DE, 16)] * attr_j
                    for cc in range(8):
                        off = h * C + cc * 16
                        acc = acc + qrows[j, pl.ds(off, 16)] * krows[j, pl.ds(off, 16)]
                    alpha = jnp.where(lanes == jnp.full((16,), h, jnp.int32),
                                      _fold_sum(acc), alpha)
                ex = jnp.where(lane8, jnp.exp(alpha), zero16)
                srcsp = _splat(sv, j)
                dstsp = _splat(dv, j)
                rec = jnp.where(lanes == jnp.full((16,), 8, jnp.int32),
                                srcsp.astype(jnp.float32), ex)
                rec = jnp.where(lanes == jnp.full((16,), 9, jnp.int32),
                                dstsp.astype(jnp.float32), rec)
                recblk[j, pl.ds(0, 16)] = rec
                recblk[j, pl.ds(16, 16)] = attr_j
                return 0

            lax.fori_loop(0, 16, edge, 0)

            zi = jnp.zeros((16,), jnp.int32)
            onei = jnp.full((16,), 1, jnp.int32)
            chv = lax.div(dv, jnp.full((16,), CH, jnp.int32))
            bases = basesr[...]
            pos = zi
            newbases = bases
            for ch in range(NCHUNK):
                chvq = jnp.full((16,), ch, jnp.int32)
                m = chv == chvq
                incl = _scan_incl(jnp.where(m, onei, zi))
                cnt = _splat(incl, 15)
                pos = pos + jnp.where(m, _splat(bases, ch) + incl - onei, zi)
                newbases = newbases + jnp.where(lanes == chvq, cnt, zi)
            basesr[...] = newbases
            posr[...] = pos
            pltpu.sync_copy(recblk, buck_hbm.at[posr])
            return 0

        lax.fori_loop(0, EPW // 16, group, 0)

        def zrb(r, _):
            recblk[r, pl.ds(0, 16)] = zero16
            recblk[r, pl.ds(16, 16)] = zero16
            return 0
        lax.fori_loop(0, 16, zrb, 0)
        pltpu.sync_copy(cse_hbm, srcv)
        pltpu.sync_copy(cpe_hbm, dstv)
        csev = srcv[...]
        cpev = dstv[...]
        zi = jnp.zeros((16,), jnp.int32)
        onei = jnp.full((16,), 1, jnp.int32)
        for ch in range(NCHUNK):
            @pl.when(wid == ch)
            def _():
                padcnt = _splat(cpev, ch) - _splat(csev, ch)
                m = jnp.clip(padcnt - lanes, zi, onei)
                trash = jnp.full((16,), NB_ROWS - 16, jnp.int32) + lanes
                pos = m * (_splat(csev, ch) + lanes) + (onei - m) * trash
                posr[...] = pos
                pltpu.sync_copy(recblk, buck_hbm.at[posr])

    return kfn(src, dst, edge_attr, q, k, qe, offs, cse, cpe)


NSL = H + 2


def _sc_numv(buck, v, csb, cpe):
    accs_t = [pltpu.VMEM_SHARED((CH, 128), jnp.float32) for _ in range(NSL)]
    msgs_t = [pltpu.VMEM((16, 128), jnp.float32) for _ in range(NSL)]

    @functools.partial(
        pl.kernel,
        out_type=jax.ShapeDtypeStruct((NSL, NCHUNK * CH, 128), jnp.float32),
        mesh=_SC_MESH(),
        scratch_types=[
            pltpu.VMEM((16, RECW), jnp.float32),
            pltpu.VMEM((16,), jnp.int32),
            pltpu.VMEM((16,), jnp.int32),
            pltpu.VMEM((16, HC), jnp.float32),
            pltpu.VMEM((8, 128), jnp.float32),
            pltpu.VMEM((32,), jnp.int32),
        ] + msgs_t + accs_t,
    )
    def kfn(buck_hbm, v_hbm, csb_hbm, cpe_hbm, numv_hbm, recs, srcir, dstlr,
            vrows, zbuf, csv, *msgacc):
        msgs = msgacc[:NSL]
        accs = msgacc[NSL:]
        core = lax.axis_index("c")
        sub = lax.axis_index("s")
        lanes = _lanes()
        zero16 = jnp.zeros((16,), jnp.float32)
        zeroi = jnp.zeros((16,), jnp.int32)

        pltpu.sync_copy(csb_hbm, csv.at[pl.ds(0, 16)])
        pltpu.sync_copy(cpe_hbm, csv.at[pl.ds(16, 16)])

        def zb(r, _):
            for cc in range(8):
                zbuf[r, pl.ds(cc * 16, 16)] = zero16
            return 0
        lax.fori_loop(0, 8, zb, 0)

        def zm(r, _):
            for cc in range(1, 8):
                msgs[H + 1][r, pl.ds(cc * 16, 16)] = zero16
            return 0
        lax.fori_loop(0, 16, zm, 0)

        starts = csv[pl.ds(0, 16)]
        ends = csv[pl.ds(16, 16)]

        for ci in range(NCHUNK // 2):
            ch = core * (NCHUNK // 2) + ci
            chlo = ch * CH
            start = pl.multiple_of(
                jnp.where(core == 0, starts[ci], starts[ci + NCHUNK // 2]), 16)
            end_s = jnp.where(core == 0, ends[ci], ends[ci + NCHUNK // 2])
            for t in range(NSL):
                for u in range(5):
                    pltpu.sync_copy(
                        zbuf, accs[t].at[pl.ds(pl.multiple_of(sub * 40 + u * 8, 8), 8)])
            plsc.subcore_barrier()

            nblocks = lax.div(end_s - start + 15, 16)
            ntrips = lax.max(lax.div(nblocks - sub + 15, 16), 0)

            def block(i, _):
                b = sub + i * 16
                s = pl.multiple_of(start + b * 16, 16)
                pltpu.sync_copy(buck_hbm.at[pl.ds(s, 16)], recs)
                sfv = jnp.zeros((16,), jnp.float32)
                dfv = jnp.zeros((16,), jnp.float32)

                def collect(j, carry):
                    sf, df = carry
                    rec_j = recs[j, pl.ds(0, 16)]
                    jq = jnp.full((16,), j, jnp.int32)
                    sf = jnp.where(lanes == jq, _splat(rec_j, 8), sf)
                    df = jnp.where(lanes == jq, _splat(rec_j, 9), df)
                    return sf, df

                sfv, dfv = lax.fori_loop(0, 16, collect, (sfv, dfv))
                svec = jnp.clip(sfv.astype(jnp.int32), zeroi,
                                jnp.full((16,), N - 1, jnp.int32))
                dloc = jnp.clip(dfv.astype(jnp.int32) - jnp.full((16,), chlo, jnp.int32),
                                zeroi, jnp.full((16,), CH - 1, jnp.int32))
                srcir[...] = svec
                dstlr[...] = dloc
                pltpu.sync_copy(v_hbm.at[srcir], vrows)

                def scale(j, _):
                    exv = recs[j, pl.ds(0, 16)]
                    attr_j = recs[j, pl.ds(16, 16)]
                    for h in range(H):
                        exh = _splat(exv, h)
                        for cc in range(8):
                            msgs[h][j, pl.ds(cc * 16, 16)] = (
                                exh * vrows[j, pl.ds(h * C + cc * 16, 16)])
                        msgs[H][j, pl.ds(h * DE, 16)] = exh * attr_j
                    msgs[H + 1][j, pl.ds(0, 16)] = exv
                    return 0

                lax.fori_loop(0, 16, scale, 0)
                for t in range(NSL):
                    pltpu.sync_copy(msgs[t], accs[t].at[dstlr], add=True)
                return 0

            lax.fori_loop(0, ntrips, block, 0)
            plsc.subcore_barrier()
            for t in range(NSL):
                pltpu.sync_copy(
                    accs[t].at[pl.ds(pl.multiple_of(sub * 40, 8), 40)],
                    numv_hbm.at[t, pl.ds(pl.multiple_of(chlo + sub * 40, 8), 40)])
            plsc.subcore_barrier()

    return kfn(buck, v, csb, cpe)


def _epi_body(numv_ref, bd2_ref, skip_ref, wd_ref, bd_ref, out_ref):
    i = pl.program_id(0)
    wea = numv_ref[H]
    emsg = jnp.dot(wea, bd2_ref[...],
                   preferred_element_type=jnp.float32)
    acc = jnp.zeros((ROWB, C), jnp.float32)
    for h in range(H):
        den_h = numv_ref[H + 1, :, h:h + 1]
        tot = numv_ref[h] + emsg[:, h * C:(h + 1) * C]
        acc = acc + tot / (den_h + 1e-16)
    node = jnp.maximum(acc * (1.0 / H) + skip_ref[...], 0.0)
    part = jnp.dot(node, wd_ref[...], preferred_element_type=jnp.float32)
    psum = jnp.sum(part).reshape(1, 1)

    @pl.when(i == 0)
    def _():
        out_ref[...] = bd_ref[...]
    out_ref[...] += psum


def _epilogue(numv, BD2, skip, Wdense, bdense):
    nblk = N // ROWB
    full = lambda shape: pl.BlockSpec(shape, lambda i: (0,) * len(shape))
    return pl.pallas_call(
        _epi_body,
        grid=(nblk,),
        in_specs=[pl.BlockSpec((NSL, ROWB, 128), lambda i: (0, i, 0)),
                  full((H * DE, HC)),
                  pl.BlockSpec((ROWB, C), lambda i: (i, 0)),
                  full((C, 1)), full((1, 1))],
        out_specs=pl.BlockSpec((1, 1), lambda i: (0, 0)),
        out_shape=jax.ShapeDtypeStruct((1, 1), jnp.float32),
    )(numv, BD2, skip, Wdense, bdense.reshape(1, 1))


def kernel(x, edge_index, edge_attr, Wq, bq, Wk, bk, Wv, bv, We,
           Wskip, bskip, Wdense, bdense):
    WeT = We.reshape(DE, H, C).transpose(1, 2, 0)
    eye = jnp.eye(H, dtype=jnp.float32)
    BD = (WeT[:, :, None, :] * eye[:, None, :, None]).reshape(HC, H * DE)
    BD2 = BD.T

    q, k, v, qe, skip = _projections(x, Wq, bq, Wk, bk, Wv, bv, Wskip, bskip, BD)
    src = edge_index[0]
    dst = edge_index[1]

    counts = _sc_count(dst)
    tot = counts.sum(0)
    ptot = ((tot + 15) // 16) * 16
    cstart_p = jnp.cumsum(ptot, dtype=jnp.int32) - ptot
    csb = cstart_p.astype(jnp.int32)
    cse = (cstart_p + tot).astype(jnp.int32)
    cpe = (cstart_p + ptot).astype(jnp.int32)
    offs = (cstart_p[None, :]
            + jnp.cumsum(counts, axis=0, dtype=jnp.int32) - counts)

    buck = _sc_edge_pass(src, dst, edge_attr, q, k, qe, offs, cse, cpe)
    numv = _sc_numv(buck, v, csb, cpe)
    out = _epilogue(numv, BD2, skip, Wdense, bdense)
    return out.reshape(1)


# speedup vs baseline: 5.2107x; 1.0652x over previous
import functools

import jax
import jax.numpy as jnp
from jax import lax
from jax.experimental import pallas as pl
from jax.experimental.pallas import tpu as pltpu
from jax.experimental.pallas import tpu_sc as plsc

N = 10000
E = 320000
D = 128
DE = 16
H = 8
C = 128
HC = H * C

ROWB = 400

NW = 32
EPW = E // NW
NCHUNK = 16
CH = 640
RECW = 128
NVW = HC + 256
NB_ROWS = E + 32 * NCHUNK + 32
_SC_MESH = lambda: plsc.VectorSubcoreMesh(core_axis_name="c", subcore_axis_name="s")


def _lanes():
    return lax.iota(jnp.int32, 16)


def _gath(vec, idx):
    return lax.gather(
        vec, idx[:, None],
        lax.GatherDimensionNumbers(offset_dims=(), collapsed_slice_dims=(0,),
                                   start_index_map=(0,)),
        (1,), mode=lax.GatherScatterMode.PROMISE_IN_BOUNDS)


def _splat(vec, i):
    return _gath(vec, jnp.full((16,), i, jnp.int32))


def _fold_sum(v):
    lanes = _lanes()
    for kk in (1, 2, 4, 8):
        v = v + _gath(v, jnp.bitwise_xor(lanes, jnp.full((16,), kk, jnp.int32)))
    return v


def _scan_incl(v):
    lanes = _lanes()
    zero = jnp.zeros((16,), v.dtype)
    for kk in (1, 2, 4, 8):
        kv = jnp.full((16,), kk, jnp.int32)
        shifted = _gath(v, jnp.maximum(lanes - kv, jnp.zeros((16,), jnp.int32)))
        v = v + jnp.where(lanes >= kv, shifted, zero)
    return v


def _sc_count(dst):
    @functools.partial(
        pl.kernel,
        out_type=jax.ShapeDtypeStruct((NW, 16), jnp.int32),
        mesh=_SC_MESH(),
        scratch_types=[pltpu.VMEM((EPW,), jnp.int32),
                       pltpu.VMEM((16,), jnp.int32)],
    )
    def k(dst_hbm, cnt_hbm, dbuf, cntv):
        wid = lax.axis_index("s") * 2 + lax.axis_index("c")
        pltpu.sync_copy(dst_hbm.at[pl.ds(wid * EPW, EPW)], dbuf)
        lanes = _lanes()
        chc = jnp.full((16,), CH, jnp.int32)
        onev = jnp.full((16,), 1, jnp.int32)
        zerov = jnp.zeros((16,), jnp.int32)

        def body(g, accs):
            chv = lax.div(dbuf[pl.ds(g * 16, 16)], chc)
            return tuple(
                accs[ch] + jnp.where(chv == jnp.full((16,), ch, jnp.int32),
                                     onev, zerov)
                for ch in range(NCHUNK))

        accs = lax.fori_loop(0, EPW // 16, body,
                             tuple(jnp.zeros((16,), jnp.int32)
                                   for _ in range(NCHUNK)))
        out = jnp.zeros((16,), jnp.int32)
        for ch in range(NCHUNK):
            tot = _fold_sum(accs[ch])
            out = jnp.where(lanes == jnp.full((16,), ch, jnp.int32), tot, out)
        cntv[...] = out
        pltpu.sync_copy(cntv, cnt_hbm.at[wid])

    return k(dst)


def _proj_body(x_ref, wq_ref, bq_ref, wk_ref, bk_ref, wv_ref, bv_ref,
               wskip_ref, bskip_ref, bd_ref,
               q_ref, k_ref, v_ref, qe_ref, skip_ref):
    xb = x_ref[...]
    inv_sqrt_c = 1.0 / (C ** 0.5)
    q = (jnp.dot(xb, wq_ref[...], preferred_element_type=jnp.float32)
         + bq_ref[...]) * inv_sqrt_c
    q_ref[...] = q
    k_ref[...] = jnp.dot(xb, wk_ref[...], preferred_element_type=jnp.float32) + bk_ref[...]
    v_ref[...] = jnp.dot(xb, wv_ref[...], preferred_element_type=jnp.float32) + bv_ref[...]
    skip_ref[...] = jnp.dot(xb, wskip_ref[...], preferred_element_type=jnp.float32) + bskip_ref[...]
    qe_ref[...] = jnp.dot(q, bd_ref[...], preferred_element_type=jnp.float32)


def _projections(x, Wq, bq, Wk, bk, Wv, bv, Wskip, bskip, BD):
    nblk = N // ROWB
    full = lambda shape: pl.BlockSpec(shape, lambda i: (0,) * len(shape))
    row = lambda w: pl.BlockSpec((ROWB, w), lambda i: (i, 0))
    return pl.pallas_call(
        _proj_body,
        grid=(nblk,),
        in_specs=[row(D), full((D, HC)), full((1, HC)), full((D, HC)),
                  full((1, HC)), full((D, HC)), full((1, HC)),
                  full((D, C)), full((1, C)), full((HC, H * DE))],
        out_specs=[row(HC), row(HC), row(HC), row(H * DE), row(C)],
        out_shape=[
            jax.ShapeDtypeStruct((N, HC), jnp.float32),
            jax.ShapeDtypeStruct((N, HC), jnp.float32),
            jax.ShapeDtypeStruct((N, HC), jnp.float32),
            jax.ShapeDtypeStruct((N, H * DE), jnp.float32),
            jax.ShapeDtypeStruct((N, C), jnp.float32),
        ],
    )(x, Wq, bq.reshape(1, HC), Wk, bk.reshape(1, HC), Wv, bv.reshape(1, HC),
      Wskip, bskip.reshape(1, C), BD)


def _sc_edge_pass(src, dst, edge_attr, q, k, qe, offs, cse, cpe):
    @functools.partial(
        pl.kernel,
        out_type=jax.ShapeDtypeStruct((NB_ROWS, RECW), jnp.float32),
        mesh=_SC_MESH(),
        scratch_types=[
            pltpu.VMEM((EPW,), jnp.int32),
            pltpu.VMEM((EPW,), jnp.int32),
            pltpu.VMEM((16,), jnp.int32),
            pltpu.VMEM((16,), jnp.int32),
            pltpu.VMEM((16, DE), jnp.float32),
            pltpu.VMEM((16, HC), jnp.float32),
            pltpu.VMEM((16, HC), jnp.float32),
            pltpu.VMEM((16, H * DE), jnp.float32),
            pltpu.VMEM((16, RECW), jnp.float32),
            pltpu.VMEM((16,), jnp.int32),
            pltpu.VMEM((16,), jnp.int32),
        ],
    )
    def kfn(src_hbm, dst_hbm, attr_hbm, q_hbm, k_hbm, qe_hbm, offs_hbm,
            cse_hbm, cpe_hbm, buck_hbm, srcs, dsts, srcv, dstv, attrb,
            qrows, krows, qerows, recblk, posr, basesr):
        core = lax.axis_index("c")
        sub = lax.axis_index("s")
        wid = sub * 2 + core
        lanes = _lanes()
        zero16 = jnp.zeros((16,), jnp.float32)

        def zr(r, _):
            for cc in range(2, RECW // 16):
                recblk[r, pl.ds(cc * 16, 16)] = zero16
            return 0
        lax.fori_loop(0, 16, zr, 0)

        pltpu.sync_copy(src_hbm.at[pl.ds(wid * EPW, EPW)], srcs)
        pltpu.sync_copy(dst_hbm.at[pl.ds(wid * EPW, EPW)], dsts)
        pltpu.sync_copy(offs_hbm.at[wid], basesr)
        lane8 = lanes < jnp.full((16,), 8, jnp.int32)

        def group(g, _):
            base = wid * EPW + g * 16
            sv = srcs[pl.ds(g * 16, 16)]
            dv = dsts[pl.ds(g * 16, 16)]
            srcv[...] = sv
            dstv[...] = dv
            pltpu.sync_copy(attr_hbm.at[pl.ds(base, 16)], attrb)
            pltpu.sync_copy(q_hbm.at[dstv], qrows)
            pltpu.sync_copy(k_hbm.at[srcv], krows)
            pltpu.sync_copy(qe_hbm.at[dstv], qerows)

            def edge(j, _):
                attr_j = attrb[j, :]
                alpha = zero16
                for h in range(H):
                    acc = qerows[j, pl.ds(h * DE, 16)] * attr_j
                    for cc in range(8):
                        off = h * C + cc * 16
                        acc = acc + qrows[j, pl.ds(off, 16)] * krows[j, pl.ds(off, 16)]
                    alpha = jnp.where(lanes == jnp.full((16,), h, jnp.int32),
                                      _fold_sum(acc), alpha)
                ex = jnp.where(lane8, jnp.exp(alpha), zero16)
                srcsp = _splat(sv, j)
                dstsp = _splat(dv, j)
                rec = jnp.where(lanes == jnp.full((16,), 8, jnp.int32),
                                srcsp.astype(jnp.float32), ex)
                rec = jnp.where(lanes == jnp.full((16,), 9, jnp.int32),
                                dstsp.astype(jnp.float32), rec)
                recblk[j, pl.ds(0, 16)] = rec
                recblk[j, pl.ds(16, 16)] = attr_j
                return 0

            lax.fori_loop(0, 16, edge, 0)

            zi = jnp.zeros((16,), jnp.int32)
            onei = jnp.full((16,), 1, jnp.int32)
            chv = lax.div(dv, jnp.full((16,), CH, jnp.int32))
            bases = basesr[...]
            pos = zi
            newbases = bases
            for ch in range(NCHUNK):
                chvq = jnp.full((16,), ch, jnp.int32)
                m = chv == chvq
                incl = _scan_incl(jnp.where(m, onei, zi))
                cnt = _splat(incl, 15)
                pos = pos + jnp.where(m, _splat(bases, ch) + incl - onei, zi)
                newbases = newbases + jnp.where(lanes == chvq, cnt, zi)
            basesr[...] = newbases
            posr[...] = pos
            pltpu.sync_copy(recblk, buck_hbm.at[posr])
            return 0

        lax.fori_loop(0, EPW // 16, group, 0)

        def zrb(r, _):
            recblk[r, pl.ds(0, 16)] = zero16
            recblk[r, pl.ds(16, 16)] = zero16
            return 0
        lax.fori_loop(0, 16, zrb, 0)
        pltpu.sync_copy(cse_hbm, srcv)
        pltpu.sync_copy(cpe_hbm, dstv)
        csev = srcv[...]
        cpev = dstv[...]
        zi = jnp.zeros((16,), jnp.int32)
        onei = jnp.full((16,), 1, jnp.int32)
        for ch in range(NCHUNK):
            @pl.when(wid == ch)
            def _():
                padcnt = _splat(cpev, ch) - _splat(csev, ch)
                trash = jnp.full((16,), NB_ROWS - 16, jnp.int32) + lanes
                for half in range(2):
                    off = jnp.full((16,), half * 16, jnp.int32)
                    m = jnp.clip(padcnt - off - lanes, zi, onei)
                    pos = (m * (_splat(csev, ch) + off + lanes)
                           + (onei - m) * trash)
                    posr[...] = pos
                    pltpu.sync_copy(recblk, buck_hbm.at[posr])

    return kfn(src, dst, edge_attr, q, k, qe, offs, cse, cpe)


NSL = H + 2


def _sc_numv(buck, v, csb, cpe):
    accs_t = [pltpu.VMEM_SHARED((CH, 128), jnp.float32) for _ in range(NSL)]
    msgs_t = [pltpu.VMEM((32, 128), jnp.float32) for _ in range(NSL)]

    @functools.partial(
        pl.kernel,
        out_type=jax.ShapeDtypeStruct((NSL, NCHUNK * CH, 128), jnp.float32),
        mesh=_SC_MESH(),
        scratch_types=[
            pltpu.VMEM((32, RECW), jnp.float32),
            pltpu.VMEM((32,), jnp.int32),
            pltpu.VMEM((32,), jnp.int32),
            pltpu.VMEM((32, HC), jnp.float32),
            pltpu.VMEM((8, 128), jnp.float32),
            pltpu.VMEM((32,), jnp.int32),
        ] + msgs_t + accs_t,
    )
    def kfn(buck_hbm, v_hbm, csb_hbm, cpe_hbm, numv_hbm, recs, srcir, dstlr,
            vrows, zbuf, csv, *msgacc):
        msgs = msgacc[:NSL]
        accs = msgacc[NSL:]
        core = lax.axis_index("c")
        sub = lax.axis_index("s")
        lanes = _lanes()
        zero16 = jnp.zeros((16,), jnp.float32)
        zeroi = jnp.zeros((16,), jnp.int32)

        pltpu.sync_copy(csb_hbm, csv.at[pl.ds(0, 16)])
        pltpu.sync_copy(cpe_hbm, csv.at[pl.ds(16, 16)])

        def zb(r, _):
            for cc in range(8):
                zbuf[r, pl.ds(cc * 16, 16)] = zero16
            return 0
        lax.fori_loop(0, 8, zb, 0)

        def zm(r, _):
            for cc in range(1, 8):
                msgs[H + 1][r, pl.ds(cc * 16, 16)] = zero16
            return 0
        lax.fori_loop(0, 32, zm, 0)

        starts = csv[pl.ds(0, 16)]
        ends = csv[pl.ds(16, 16)]

        for ci in range(NCHUNK // 2):
            ch = core * (NCHUNK // 2) + ci
            chlo = ch * CH
            start = pl.multiple_of(
                jnp.where(core == 0, starts[ci], starts[ci + NCHUNK // 2]), 16)
            end_s = jnp.where(core == 0, ends[ci], ends[ci + NCHUNK // 2])
            for t in range(NSL):
                for u in range(5):
                    pltpu.sync_copy(
                        zbuf, accs[t].at[pl.ds(pl.multiple_of(sub * 40 + u * 8, 8), 8)])
            plsc.subcore_barrier()

            nblocks = lax.div(end_s - start + 31, 32)
            ntrips = lax.max(lax.div(nblocks - sub + 15, 16), 0)

            def block(i, _):
                b = sub + i * 16
                s = pl.multiple_of(start + b * 32, 16)
                pltpu.sync_copy(buck_hbm.at[pl.ds(s, 32)], recs)

                def collect(j, carry):
                    sf, df = carry
                    rec_j = recs[j, pl.ds(0, 16)]
                    jq = lax.rem(jnp.full((16,), j, jnp.int32),
                                 jnp.full((16,), 16, jnp.int32))
                    sf = jnp.where(lanes == jq, _splat(rec_j, 8), sf)
                    df = jnp.where(lanes == jq, _splat(rec_j, 9), df)
                    return sf, df

                for half in range(2):
                    sfv, dfv = lax.fori_loop(
                        half * 16, half * 16 + 16, collect,
                        (jnp.zeros((16,), jnp.float32),
                         jnp.zeros((16,), jnp.float32)))
                    svec = jnp.clip(sfv.astype(jnp.int32), zeroi,
                                    jnp.full((16,), N - 1, jnp.int32))
                    dloc = jnp.clip(
                        dfv.astype(jnp.int32) - jnp.full((16,), chlo, jnp.int32),
                        zeroi, jnp.full((16,), CH - 1, jnp.int32))
                    srcir[pl.ds(half * 16, 16)] = svec
                    dstlr[pl.ds(half * 16, 16)] = dloc
                pltpu.sync_copy(v_hbm.at[srcir], vrows)

                def scale(j, _):
                    exv = recs[j, pl.ds(0, 16)]
                    attr_j = recs[j, pl.ds(16, 16)]
                    for h in range(H):
                        exh = _splat(exv, h)
                        for cc in range(8):
                            msgs[h][j, pl.ds(cc * 16, 16)] = (
                                exh * vrows[j, pl.ds(h * C + cc * 16, 16)])
                        msgs[H][j, pl.ds(h * DE, 16)] = exh * attr_j
                    msgs[H + 1][j, pl.ds(0, 16)] = exv
                    return 0

                lax.fori_loop(0, 32, scale, 0)
                for t in range(NSL):
                    pltpu.sync_copy(msgs[t], accs[t].at[dstlr], add=True)
                return 0

            lax.fori_loop(0, ntrips, block, 0)
            plsc.subcore_barrier()
            for t in range(NSL):
                pltpu.sync_copy(
                    accs[t].at[pl.ds(pl.multiple_of(sub * 40, 8), 40)],
                    numv_hbm.at[t, pl.ds(pl.multiple_of(chlo + sub * 40, 8), 40)])
            plsc.subcore_barrier()

    return kfn(buck, v, csb, cpe)


def _epi_body(numv_ref, bd2_ref, skip_ref, wd_ref, bd_ref, out_ref):
    i = pl.program_id(0)
    wea = numv_ref[H]
    emsg = jnp.dot(wea, bd2_ref[...],
                   preferred_element_type=jnp.float32)
    acc = jnp.zeros((ROWB, C), jnp.float32)
    for h in range(H):
        den_h = numv_ref[H + 1, :, h:h + 1]
        tot = numv_ref[h] + emsg[:, h * C:(h + 1) * C]
        acc = acc + tot / (den_h + 1e-16)
    node = jnp.maximum(acc * (1.0 / H) + skip_ref[...], 0.0)
    part = jnp.dot(node, wd_ref[...], preferred_element_type=jnp.float32)
    psum = jnp.sum(part).reshape(1, 1)

    @pl.when(i == 0)
    def _():
        out_ref[...] = bd_ref[...]
    out_ref[...] += psum


def _epilogue(numv, BD2, skip, Wdense, bdense):
    nblk = N // ROWB
    full = lambda shape: pl.BlockSpec(shape, lambda i: (0,) * len(shape))
    return pl.pallas_call(
        _epi_body,
        grid=(nblk,),
        in_specs=[pl.BlockSpec((NSL, ROWB, 128), lambda i: (0, i, 0)),
                  full((H * DE, HC)),
                  pl.BlockSpec((ROWB, C), lambda i: (i, 0)),
                  full((C, 1)), full((1, 1))],
        out_specs=pl.BlockSpec((1, 1), lambda i: (0, 0)),
        out_shape=jax.ShapeDtypeStruct((1, 1), jnp.float32),
    )(numv, BD2, skip, Wdense, bdense.reshape(1, 1))


def kernel(x, edge_index, edge_attr, Wq, bq, Wk, bk, Wv, bv, We,
           Wskip, bskip, Wdense, bdense):
    WeT = We.reshape(DE, H, C).transpose(1, 2, 0)
    eye = jnp.eye(H, dtype=jnp.float32)
    BD = (WeT[:, :, None, :] * eye[:, None, :, None]).reshape(HC, H * DE)
    BD2 = BD.T

    q, k, v, qe, skip = _projections(x, Wq, bq, Wk, bk, Wv, bv, Wskip, bskip, BD)
    src = edge_index[0]
    dst = edge_index[1]

    counts = _sc_count(dst)
    tot = counts.sum(0)
    ptot = ((tot + 31) // 32) * 32
    cstart_p = jnp.cumsum(ptot, dtype=jnp.int32) - ptot
    csb = cstart_p.astype(jnp.int32)
    cse = (cstart_p + tot).astype(jnp.int32)
    cpe = (cstart_p + ptot).astype(jnp.int32)
    offs = (cstart_p[None, :]
            + jnp.cumsum(counts, axis=0, dtype=jnp.int32) - counts)

    buck = _sc_edge_pass(src, dst, edge_attr, q, k, qe, offs, cse, cpe)
    numv = _sc_numv(buck, v, csb, cpe)
    out = _epilogue(numv, BD2, skip, Wdense, bdense)
    return out.reshape(1)


# speedup vs baseline: 6.1725x; 1.1846x over previous
import functools

import jax
import jax.numpy as jnp
from jax import lax
from jax.experimental import pallas as pl
from jax.experimental.pallas import tpu as pltpu
from jax.experimental.pallas import tpu_sc as plsc

N = 10000
E = 320000
D = 128
DE = 16
H = 8
C = 128
HC = H * C

ROWB = 400

NW = 32
EPW = E // NW
NCHUNK = 16
CH = 640
RECW = 128
NVW = HC + 256
NB_ROWS = E + 32 * NCHUNK + 32
_SC_MESH = lambda: plsc.VectorSubcoreMesh(core_axis_name="c", subcore_axis_name="s")


def _lanes():
    return lax.iota(jnp.int32, 16)


def _gath(vec, idx):
    return lax.gather(
        vec, idx[:, None],
        lax.GatherDimensionNumbers(offset_dims=(), collapsed_slice_dims=(0,),
                                   start_index_map=(0,)),
        (1,), mode=lax.GatherScatterMode.PROMISE_IN_BOUNDS)


def _splat(vec, i):
    return _gath(vec, jnp.full((16,), i, jnp.int32))


def _fold_sum(v):
    lanes = _lanes()
    for kk in (1, 2, 4, 8):
        v = v + _gath(v, jnp.bitwise_xor(lanes, jnp.full((16,), kk, jnp.int32)))
    return v


def _scan_incl(v):
    lanes = _lanes()
    zero = jnp.zeros((16,), v.dtype)
    for kk in (1, 2, 4, 8):
        kv = jnp.full((16,), kk, jnp.int32)
        shifted = _gath(v, jnp.maximum(lanes - kv, jnp.zeros((16,), jnp.int32)))
        v = v + jnp.where(lanes >= kv, shifted, zero)
    return v


def _sc_count(dst):
    @functools.partial(
        pl.kernel,
        out_type=jax.ShapeDtypeStruct((NW, 16), jnp.int32),
        mesh=_SC_MESH(),
        scratch_types=[pltpu.VMEM((EPW,), jnp.int32),
                       pltpu.VMEM((16,), jnp.int32)],
    )
    def k(dst_hbm, cnt_hbm, dbuf, cntv):
        wid = lax.axis_index("s") * 2 + lax.axis_index("c")
        pltpu.sync_copy(dst_hbm.at[pl.ds(wid * EPW, EPW)], dbuf)
        lanes = _lanes()
        chc = jnp.full((16,), CH, jnp.int32)
        onev = jnp.full((16,), 1, jnp.int32)
        zerov = jnp.zeros((16,), jnp.int32)

        def body(g, accs):
            chv = lax.div(dbuf[pl.ds(g * 16, 16)], chc)
            return tuple(
                accs[ch] + jnp.where(chv == jnp.full((16,), ch, jnp.int32),
                                     onev, zerov)
                for ch in range(NCHUNK))

        accs = lax.fori_loop(0, EPW // 16, body,
                             tuple(jnp.zeros((16,), jnp.int32)
                                   for _ in range(NCHUNK)))
        out = jnp.zeros((16,), jnp.int32)
        for ch in range(NCHUNK):
            tot = _fold_sum(accs[ch])
            out = jnp.where(lanes == jnp.full((16,), ch, jnp.int32), tot, out)
        cntv[...] = out
        pltpu.sync_copy(cntv, cnt_hbm.at[wid])

    return k(dst)


def _proj_body(x_ref, wq_ref, bq_ref, wk_ref, bk_ref, wv_ref, bv_ref,
               wskip_ref, bskip_ref, bd_ref,
               q_ref, k_ref, v_ref, qe_ref, skip_ref):
    xb = x_ref[...]
    inv_sqrt_c = 1.0 / (C ** 0.5)
    q = (jnp.dot(xb, wq_ref[...], preferred_element_type=jnp.float32)
         + bq_ref[...]) * inv_sqrt_c
    q_ref[...] = q
    k_ref[...] = jnp.dot(xb, wk_ref[...], preferred_element_type=jnp.float32) + bk_ref[...]
    v_ref[...] = jnp.dot(xb, wv_ref[...], preferred_element_type=jnp.float32) + bv_ref[...]
    skip_ref[...] = jnp.dot(xb, wskip_ref[...], preferred_element_type=jnp.float32) + bskip_ref[...]
    qe_ref[...] = jnp.dot(q, bd_ref[...], preferred_element_type=jnp.float32)


def _projections(x, Wq, bq, Wk, bk, Wv, bv, Wskip, bskip, BD):
    nblk = N // ROWB
    full = lambda shape: pl.BlockSpec(shape, lambda i: (0,) * len(shape))
    row = lambda w: pl.BlockSpec((ROWB, w), lambda i: (i, 0))
    return pl.pallas_call(
        _proj_body,
        grid=(nblk,),
        in_specs=[row(D), full((D, HC)), full((1, HC)), full((D, HC)),
                  full((1, HC)), full((D, HC)), full((1, HC)),
                  full((D, C)), full((1, C)), full((HC, H * DE))],
        out_specs=[row(HC), row(HC), row(HC), row(H * DE), row(C)],
        out_shape=[
            jax.ShapeDtypeStruct((N, HC), jnp.float32),
            jax.ShapeDtypeStruct((N, HC), jnp.float32),
            jax.ShapeDtypeStruct((N, HC), jnp.float32),
            jax.ShapeDtypeStruct((N, H * DE), jnp.float32),
            jax.ShapeDtypeStruct((N, C), jnp.float32),
        ],
    )(x, Wq, bq.reshape(1, HC), Wk, bk.reshape(1, HC), Wv, bv.reshape(1, HC),
      Wskip, bskip.reshape(1, C), BD)


def _sc_edge_pass(src, dst, edge_attr, q, k, qe, offs, cse, cpe):
    @functools.partial(
        pl.kernel,
        out_type=jax.ShapeDtypeStruct((NB_ROWS, RECW), jnp.float32),
        mesh=_SC_MESH(),
        scratch_types=[
            pltpu.VMEM((EPW,), jnp.int32),
            pltpu.VMEM((EPW,), jnp.int32),
            pltpu.VMEM((16,), jnp.int32),
            pltpu.VMEM((16,), jnp.int32),
            pltpu.VMEM((16, DE), jnp.float32),
            pltpu.VMEM((16, HC), jnp.float32),
            pltpu.VMEM((16, HC), jnp.float32),
            pltpu.VMEM((16, H * DE), jnp.float32),
            pltpu.VMEM((16, RECW), jnp.float32),
            pltpu.VMEM((16,), jnp.int32),
            pltpu.VMEM((16,), jnp.int32),
            pltpu.SemaphoreType.DMA,
        ],
    )
    def kfn(src_hbm, dst_hbm, attr_hbm, q_hbm, k_hbm, qe_hbm, offs_hbm,
            cse_hbm, cpe_hbm, buck_hbm, srcs, dsts, srcv, dstv, attrb,
            qrows, krows, qerows, recblk, posr, basesr, gsem):
        core = lax.axis_index("c")
        sub = lax.axis_index("s")
        wid = sub * 2 + core
        lanes = _lanes()
        zero16 = jnp.zeros((16,), jnp.float32)

        def zr(r, _):
            for cc in range(2, RECW // 16):
                recblk[r, pl.ds(cc * 16, 16)] = zero16
            return 0
        lax.fori_loop(0, 16, zr, 0)

        pltpu.sync_copy(src_hbm.at[pl.ds(wid * EPW, EPW)], srcs)
        pltpu.sync_copy(dst_hbm.at[pl.ds(wid * EPW, EPW)], dsts)
        pltpu.sync_copy(offs_hbm.at[wid], basesr)
        lane8 = lanes < jnp.full((16,), 8, jnp.int32)

        def group(g, _):
            base = wid * EPW + g * 16
            sv = srcs[pl.ds(g * 16, 16)]
            dv = dsts[pl.ds(g * 16, 16)]
            srcv[...] = sv
            dstv[...] = dv
            h0 = pltpu.async_copy(attr_hbm.at[pl.ds(base, 16)], attrb, gsem)
            h1 = pltpu.async_copy(q_hbm.at[dstv], qrows, gsem)
            h2 = pltpu.async_copy(k_hbm.at[srcv], krows, gsem)
            h3 = pltpu.async_copy(qe_hbm.at[dstv], qerows, gsem)
            h0.wait()
            h1.wait()
            h2.wait()
            h3.wait()

            def edge(j, _):
                attr_j = attrb[j, :]
                alpha = zero16
                for h in range(H):
                    acc = qerows[j, pl.ds(h * DE, 16)] * attr_j
                    for cc in range(8):
                        off = h * C + cc * 16
                        acc = acc + qrows[j, pl.ds(off, 16)] * krows[j, pl.ds(off, 16)]
                    alpha = jnp.where(lanes == jnp.full((16,), h, jnp.int32),
                                      _fold_sum(acc), alpha)
                ex = jnp.where(lane8, jnp.exp(alpha), zero16)
                srcsp = _splat(sv, j)
                dstsp = _splat(dv, j)
                rec = jnp.where(lanes == jnp.full((16,), 8, jnp.int32),
                                srcsp.astype(jnp.float32), ex)
                rec = jnp.where(lanes == jnp.full((16,), 9, jnp.int32),
                                dstsp.astype(jnp.float32), rec)
                recblk[j, pl.ds(0, 16)] = rec
                recblk[j, pl.ds(16, 16)] = attr_j
                return 0

            lax.fori_loop(0, 16, edge, 0)

            zi = jnp.zeros((16,), jnp.int32)
            onei = jnp.full((16,), 1, jnp.int32)
            chv = lax.div(dv, jnp.full((16,), CH, jnp.int32))
            bases = basesr[...]
            pos = zi
            newbases = bases
            for ch in range(NCHUNK):
                chvq = jnp.full((16,), ch, jnp.int32)
                m = chv == chvq
                incl = _scan_incl(jnp.where(m, onei, zi))
                cnt = _splat(incl, 15)
                pos = pos + jnp.where(m, _splat(bases, ch) + incl - onei, zi)
                newbases = newbases + jnp.where(lanes == chvq, cnt, zi)
            basesr[...] = newbases
            posr[...] = pos
            pltpu.sync_copy(recblk, buck_hbm.at[posr])
            return 0

        lax.fori_loop(0, EPW // 16, group, 0)

        def zrb(r, _):
            recblk[r, pl.ds(0, 16)] = zero16
            recblk[r, pl.ds(16, 16)] = zero16
            return 0
        lax.fori_loop(0, 16, zrb, 0)
        pltpu.sync_copy(cse_hbm, srcv)
        pltpu.sync_copy(cpe_hbm, dstv)
        csev = srcv[...]
        cpev = dstv[...]
        zi = jnp.zeros((16,), jnp.int32)
        onei = jnp.full((16,), 1, jnp.int32)
        for ch in range(NCHUNK):
            @pl.when(wid == ch)
            def _():
                padcnt = _splat(cpev, ch) - _splat(csev, ch)
                trash = jnp.full((16,), NB_ROWS - 16, jnp.int32) + lanes
                for half in range(2):
                    off = jnp.full((16,), half * 16, jnp.int32)
                    m = jnp.clip(padcnt - off - lanes, zi, onei)
                    pos = (m * (_splat(csev, ch) + off + lanes)
                           + (onei - m) * trash)
                    posr[...] = pos
                    pltpu.sync_copy(recblk, buck_hbm.at[posr])

    return kfn(src, dst, edge_attr, q, k, qe, offs, cse, cpe)


NSL = H + 2


def _sc_numv(buck, v, csb, cpe):
    accs_t = [pltpu.VMEM_SHARED((CH, 128), jnp.float32) for _ in range(NSL)]
    msgs_t = [pltpu.VMEM((32, 128), jnp.float32) for _ in range(NSL)]

    @functools.partial(
        pl.kernel,
        out_type=jax.ShapeDtypeStruct((NSL, NCHUNK * CH, 128), jnp.float32),
        mesh=_SC_MESH(),
        scratch_types=[
            pltpu.VMEM((32, RECW), jnp.float32),
            pltpu.VMEM((32,), jnp.int32),
            pltpu.VMEM((32,), jnp.int32),
            pltpu.VMEM((32, HC), jnp.float32),
            pltpu.VMEM((8, 128), jnp.float32),
            pltpu.VMEM((32,), jnp.int32),
            pltpu.SemaphoreType.DMA,
            pltpu.SemaphoreType.DMA,
        ] + msgs_t + accs_t,
    )
    def kfn(buck_hbm, v_hbm, csb_hbm, cpe_hbm, numv_hbm, recs, srcir, dstlr,
            vrows, zbuf, csv, scsem, gsem, *msgacc):
        msgs = msgacc[:NSL]
        accs = msgacc[NSL:]
        core = lax.axis_index("c")
        sub = lax.axis_index("s")
        lanes = _lanes()
        zero16 = jnp.zeros((16,), jnp.float32)
        zeroi = jnp.zeros((16,), jnp.int32)

        pltpu.sync_copy(csb_hbm, csv.at[pl.ds(0, 16)])
        pltpu.sync_copy(cpe_hbm, csv.at[pl.ds(16, 16)])

        def zb(r, _):
            for cc in range(8):
                zbuf[r, pl.ds(cc * 16, 16)] = zero16
            return 0
        lax.fori_loop(0, 8, zb, 0)

        def zm(r, _):
            for cc in range(1, 8):
                msgs[H + 1][r, pl.ds(cc * 16, 16)] = zero16
            return 0
        lax.fori_loop(0, 32, zm, 0)

        starts = csv[pl.ds(0, 16)]
        ends = csv[pl.ds(16, 16)]

        for ci in range(NCHUNK // 2):
            ch = core * (NCHUNK // 2) + ci
            chlo = ch * CH
            start = pl.multiple_of(
                jnp.where(core == 0, starts[ci], starts[ci + NCHUNK // 2]), 16)
            end_s = jnp.where(core == 0, ends[ci], ends[ci + NCHUNK // 2])
            for t in range(NSL):
                for u in range(5):
                    pltpu.sync_copy(
                        zbuf, accs[t].at[pl.ds(pl.multiple_of(sub * 40 + u * 8, 8), 8)])
            plsc.subcore_barrier()

            nblocks = lax.div(end_s - start + 31, 32)
            ntrips = lax.max(lax.div(nblocks - sub + 15, 16), 0)

            def block(i, _):
                b = sub + i * 16
                s = pl.multiple_of(start + b * 32, 16)
                pltpu.sync_copy(buck_hbm.at[pl.ds(s, 32)], recs)

                def collect(j, carry):
                    sf, df = carry
                    rec_j = recs[j, pl.ds(0, 16)]
                    jq = lax.rem(jnp.full((16,), j, jnp.int32),
                                 jnp.full((16,), 16, jnp.int32))
                    sf = jnp.where(lanes == jq, _splat(rec_j, 8), sf)
                    df = jnp.where(lanes == jq, _splat(rec_j, 9), df)
                    return sf, df

                for half in range(2):
                    sfv, dfv = lax.fori_loop(
                        half * 16, half * 16 + 16, collect,
                        (jnp.zeros((16,), jnp.float32),
                         jnp.zeros((16,), jnp.float32)))
                    svec = jnp.clip(sfv.astype(jnp.int32), zeroi,
                                    jnp.full((16,), N - 1, jnp.int32))
                    dloc = jnp.clip(
                        dfv.astype(jnp.int32) - jnp.full((16,), chlo, jnp.int32),
                        zeroi, jnp.full((16,), CH - 1, jnp.int32))
                    srcir[pl.ds(half * 16, 16)] = svec
                    dstlr[pl.ds(half * 16, 16)] = dloc
                pltpu.sync_copy(v_hbm.at[srcir], vrows)

                def scale(j, _):
                    exv = recs[j, pl.ds(0, 16)]
                    attr_j = recs[j, pl.ds(16, 16)]
                    for h in range(H):
                        exh = _splat(exv, h)
                        for cc in range(8):
                            msgs[h][j, pl.ds(cc * 16, 16)] = (
                                exh * vrows[j, pl.ds(h * C + cc * 16, 16)])
                        msgs[H][j, pl.ds(h * DE, 16)] = exh * attr_j
                    msgs[H + 1][j, pl.ds(0, 16)] = exv
                    return 0

                lax.fori_loop(0, 32, scale, 0)
                handles = [pltpu.async_copy(msgs[t], accs[t].at[dstlr], scsem,
                                            add=True)
                           for t in range(NSL)]
                for hdl in handles:
                    hdl.wait()
                return 0

            lax.fori_loop(0, ntrips, block, 0)
            plsc.subcore_barrier()
            for t in range(NSL):
                pltpu.sync_copy(
                    accs[t].at[pl.ds(pl.multiple_of(sub * 40, 8), 40)],
                    numv_hbm.at[t, pl.ds(pl.multiple_of(chlo + sub * 40, 8), 40)])
            plsc.subcore_barrier()

    return kfn(buck, v, csb, cpe)


def _epi_body(numv_ref, bd2_ref, skip_ref, wd_ref, bd_ref, out_ref):
    i = pl.program_id(0)
    wea = numv_ref[H]
    emsg = jnp.dot(wea, bd2_ref[...],
                   preferred_element_type=jnp.float32)
    acc = jnp.zeros((ROWB, C), jnp.float32)
    for h in range(H):
        den_h = numv_ref[H + 1, :, h:h + 1]
        tot = numv_ref[h] + emsg[:, h * C:(h + 1) * C]
        acc = acc + tot / (den_h + 1e-16)
    node = jnp.maximum(acc * (1.0 / H) + skip_ref[...], 0.0)
    part = jnp.dot(node, wd_ref[...], preferred_element_type=jnp.float32)
    psum = jnp.sum(part).reshape(1, 1)

    @pl.when(i == 0)
    def _():
        out_ref[...] = bd_ref[...]
    out_ref[...] += psum


def _epilogue(numv, BD2, skip, Wdense, bdense):
    nblk = N // ROWB
    full = lambda shape: pl.BlockSpec(shape, lambda i: (0,) * len(shape))
    return pl.pallas_call(
        _epi_body,
        grid=(nblk,),
        in_specs=[pl.BlockSpec((NSL, ROWB, 128), lambda i: (0, i, 0)),
                  full((H * DE, HC)),
                  pl.BlockSpec((ROWB, C), lambda i: (i, 0)),
                  full((C, 1)), full((1, 1))],
        out_specs=pl.BlockSpec((1, 1), lambda i: (0, 0)),
        out_shape=jax.ShapeDtypeStruct((1, 1), jnp.float32),
    )(numv, BD2, skip, Wdense, bdense.reshape(1, 1))


def kernel(x, edge_index, edge_attr, Wq, bq, Wk, bk, Wv, bv, We,
           Wskip, bskip, Wdense, bdense):
    WeT = We.reshape(DE, H, C).transpose(1, 2, 0)
    eye = jnp.eye(H, dtype=jnp.float32)
    BD = (WeT[:, :, None, :] * eye[:, None, :, None]).reshape(HC, H * DE)
    BD2 = BD.T

    q, k, v, qe, skip = _projections(x, Wq, bq, Wk, bk, Wv, bv, Wskip, bskip, BD)
    src = edge_index[0]
    dst = edge_index[1]

    counts = _sc_count(dst)
    tot = counts.sum(0)
    ptot = ((tot + 31) // 32) * 32
    cstart_p = jnp.cumsum(ptot, dtype=jnp.int32) - ptot
    csb = cstart_p.astype(jnp.int32)
    cse = (cstart_p + tot).astype(jnp.int32)
    cpe = (cstart_p + ptot).astype(jnp.int32)
    offs = (cstart_p[None, :]
            + jnp.cumsum(counts, axis=0, dtype=jnp.int32) - counts)

    buck = _sc_edge_pass(src, dst, edge_attr, q, k, qe, offs, cse, cpe)
    numv = _sc_numv(buck, v, csb, cpe)
    out = _epilogue(numv, BD2, skip, Wdense, bdense)
    return out.reshape(1)
